# baseline (speedup 1.0000x reference)
import functools

import jax
import jax.numpy as jnp
from jax import lax
from jax.experimental import pallas as pl
from jax.experimental.pallas import tpu as pltpu
from jax.experimental.pallas import tpu_sc as plsc

D = 64
B = 16384
NNEG = 5
T = 2 + NNEG

_info = plsc.get_sparse_core_info()
_NC, _NS, _L = _info.num_cores, _info.num_subcores, _info.num_lanes
NW = _NC * _NS
RW = B // NW
_GROUPS = RW // _L

_mesh = plsc.VectorSubcoreMesh(core_axis_name="c", subcore_axis_name="s")


@functools.partial(
    pl.kernel,
    mesh=_mesh,
    out_type=jax.ShapeDtypeStruct((T - 1, B), jnp.float32),
    scratch_types=[
        pltpu.VMEM((T * RW,), jnp.int32),
        pltpu.VMEM((RW, D), jnp.float32),
        pltpu.VMEM((RW, D), jnp.float32),
        pltpu.VMEM((RW, D), jnp.float32),
        pltpu.VMEM((T - 1, RW), jnp.float32),
        pltpu.SemaphoreType.DMA,
        pltpu.SemaphoreType.DMA,
        pltpu.SemaphoreType.DMA,
    ],
    compiler_params=pltpu.CompilerParams(
        use_tc_tiling_on_sc=False, needs_layout_passes=False),
)
def _sc_scores(emb_hbm, vemb_hbm, ids_hbm, out_hbm,
               idx_v, wbuf, vbuf0, vbuf1, sbuf, gw, g0, g1):
    wid = lax.axis_index("s") * _NC + lax.axis_index("c")
    vbufs = (vbuf0, vbuf1)
    gsems = (g0, g1)

    pltpu.sync_copy(ids_hbm.at[wid], idx_v)

    w_cp = pltpu.async_copy(emb_hbm.at[idx_v.at[pl.ds(0, RW)]], wbuf, gw)
    gathers = [None, None]
    gathers[0] = pltpu.async_copy(
        vemb_hbm.at[idx_v.at[pl.ds(RW, RW)]], vbufs[0], gsems[0])
    w_cp.wait()

    def compute_stream(t, vb):
        def group_body(g, _):
            rows = lax.iota(jnp.int32, _L) + g * _L
            acc = jnp.zeros((_L,), jnp.float32)
            cols0 = lax.iota(jnp.int32, _L)
            for d in range(D):
                cols = (cols0 + d) & (D - 1)
                wcol = plsc.load_gather(wbuf, [rows, cols])
                vcol = plsc.load_gather(vb, [rows, cols])
                acc = acc + wcol * vcol
            sbuf[t - 1, pl.ds(g * _L, _L)] = acc
            return 0

        lax.fori_loop(0, _GROUPS, group_body, 0)

    for t in range(1, T):
        cur = (t - 1) % 2
        if t + 1 < T:
            gathers[t % 2] = pltpu.async_copy(
                vemb_hbm.at[idx_v.at[pl.ds((t + 1) * RW, RW)]],
                vbufs[t % 2], gsems[t % 2])
        gathers[cur].wait()
        compute_stream(t, vbufs[cur])

    pltpu.sync_copy(sbuf, out_hbm.at[:, pl.ds(wid * RW, RW)])


_TW = 512
_TGRID = -(-1000000 // _TW)


def _transpose_body(x_ref, o_ref):
    o_ref[...] = x_ref[...].T


_tc_rowmajor = pl.pallas_call(
    _transpose_body,
    grid=(_TGRID,),
    in_specs=[pl.BlockSpec((D, _TW), lambda i: (0, i))],
    out_specs=pl.BlockSpec((_TW, D), lambda i: (i, 0)),
    out_shape=jax.ShapeDtypeStruct((1000000, D), jnp.float32),
)


def _loss_body(s_ref, out_ref):
    s = jnp.clip(s_ref[...], -10.0, 10.0)
    pos = s[0:1]
    neg = s[1:]
    tot = jnp.sum(jnp.log1p(jnp.exp(-pos))) + jnp.sum(jnp.log1p(jnp.exp(neg)))
    out_ref[...] = jnp.reshape(tot, (1, 1))


_loss = pl.pallas_call(
    _loss_body,
    out_shape=jax.ShapeDtypeStruct((1, 1), jnp.float32),
)


def kernel(embedding, v_embedding, wids, vids, neg_vids):
    ids = jnp.concatenate([
        wids.astype(jnp.int32),
        vids.astype(jnp.int32),
        neg_vids.T.astype(jnp.int32).reshape(-1),
    ])
    ids = ids.reshape(T, NW, RW).transpose(1, 0, 2).reshape(NW, -1)
    emb_rm = _tc_rowmajor(embedding.T)
    scores = _sc_scores(emb_rm, v_embedding, ids)
    return _loss(scores)[0, 0] / B


# speedup vs baseline: 1.1706x; 1.1706x over previous
import functools

import jax
import jax.numpy as jnp
from jax import lax
from jax.experimental import pallas as pl
from jax.experimental.pallas import tpu as pltpu
from jax.experimental.pallas import tpu_sc as plsc

D = 64
B = 16384
NNEG = 5
T = 2 + NNEG

_info = plsc.get_sparse_core_info()
_NC, _NS, _L = _info.num_cores, _info.num_subcores, _info.num_lanes
NW = _NC * _NS
RW = B // NW
_GROUPS = RW // _L

_mesh = plsc.VectorSubcoreMesh(core_axis_name="c", subcore_axis_name="s")


@functools.partial(
    pl.kernel,
    mesh=_mesh,
    out_type=jax.ShapeDtypeStruct((T - 1, B), jnp.float32),
    scratch_types=[
        pltpu.VMEM((T * RW,), jnp.int32),
        pltpu.VMEM((RW, D), jnp.float32),
        pltpu.VMEM((RW, D), jnp.float32),
        pltpu.VMEM((RW, D), jnp.float32),
        pltpu.VMEM((T - 1, RW), jnp.float32),
        pltpu.SemaphoreType.DMA,
        pltpu.SemaphoreType.DMA,
        pltpu.SemaphoreType.DMA,
    ],
    compiler_params=pltpu.CompilerParams(
        use_tc_tiling_on_sc=False, needs_layout_passes=False),
)
def _sc_scores(emb_hbm, vemb_hbm, ids_hbm, out_hbm,
               idx_v, wbuf, vbuf0, vbuf1, sbuf, gw, g0, g1):
    wid = lax.axis_index("s") * _NC + lax.axis_index("c")
    vbufs = (vbuf0, vbuf1)
    gsems = (g0, g1)

    pltpu.sync_copy(ids_hbm.at[wid], idx_v)

    w_cp = pltpu.async_copy(emb_hbm.at[idx_v.at[pl.ds(0, RW)]], wbuf, gw)
    gathers = [None, None]
    gathers[0] = pltpu.async_copy(
        vemb_hbm.at[idx_v.at[pl.ds(RW, RW)]], vbufs[0], gsems[0])
    w_cp.wait()

    def compute_stream(t, vb):
        def group_body(g, _):
            rows = lax.iota(jnp.int32, _L) + g * _L
            acc = jnp.zeros((_L,), jnp.float32)
            cols0 = lax.iota(jnp.int32, _L)
            for d in range(D):
                cols = (cols0 + d) & (D - 1)
                wcol = plsc.load_gather(wbuf, [rows, cols])
                vcol = plsc.load_gather(vb, [rows, cols])
                acc = acc + wcol * vcol
            sbuf[t - 1, pl.ds(g * _L, _L)] = acc
            return 0

        lax.fori_loop(0, _GROUPS, group_body, 0)

    for t in range(1, T):
        cur = (t - 1) % 2
        if t + 1 < T:
            gathers[t % 2] = pltpu.async_copy(
                vemb_hbm.at[idx_v.at[pl.ds((t + 1) * RW, RW)]],
                vbufs[t % 2], gsems[t % 2])
        gathers[cur].wait()
        compute_stream(t, vbufs[cur])

    pltpu.sync_copy(sbuf, out_hbm.at[:, pl.ds(wid * RW, RW)])


_TW = 2048
_TGRID = -(-1000000 // _TW)


def _transpose_body(x_ref, o_ref):
    eye = jnp.eye(D, dtype=jnp.float32)
    o_ref[...] = jax.lax.dot_general(
        x_ref[...], eye, (((0,), (0,)), ((), ())),
        preferred_element_type=jnp.float32)


_tc_rowmajor = pl.pallas_call(
    _transpose_body,
    grid=(_TGRID,),
    in_specs=[pl.BlockSpec((D, _TW), lambda i: (0, i))],
    out_specs=pl.BlockSpec((_TW, D), lambda i: (i, 0)),
    out_shape=jax.ShapeDtypeStruct((1000000, D), jnp.float32),
)


def _loss_body(s_ref, out_ref):
    s = jnp.clip(s_ref[...], -10.0, 10.0)
    pos = s[0:1]
    neg = s[1:]
    tot = jnp.sum(jnp.log1p(jnp.exp(-pos))) + jnp.sum(jnp.log1p(jnp.exp(neg)))
    out_ref[...] = jnp.reshape(tot, (1, 1))


_loss = pl.pallas_call(
    _loss_body,
    out_shape=jax.ShapeDtypeStruct((1, 1), jnp.float32),
)


def kernel(embedding, v_embedding, wids, vids, neg_vids):
    ids = jnp.concatenate([
        wids.astype(jnp.int32),
        vids.astype(jnp.int32),
        neg_vids.T.astype(jnp.int32).reshape(-1),
    ])
    ids = ids.reshape(T, NW, RW).transpose(1, 0, 2).reshape(NW, -1)
    emb_rm = _tc_rowmajor(embedding.T)
    vemb_rm = _tc_rowmajor(v_embedding.T)
    scores = _sc_scores(emb_rm, vemb_rm, ids)
    return _loss(scores)[0, 0] / B


# speedup vs baseline: 1.5430x; 1.3181x over previous
import functools

import jax
import jax.numpy as jnp
from jax import lax
from jax.experimental import pallas as pl
from jax.experimental.pallas import tpu as pltpu
from jax.experimental.pallas import tpu_sc as plsc

D = 64
B = 16384
NNEG = 5
T = 2 + NNEG

_info = plsc.get_sparse_core_info()
_NC, _NS, _L = _info.num_cores, _info.num_subcores, _info.num_lanes
NW = _NC * _NS
RW = B // NW
_GROUPS = RW // _L

_mesh = plsc.VectorSubcoreMesh(core_axis_name="c", subcore_axis_name="s")


@functools.partial(
    pl.kernel,
    mesh=_mesh,
    out_type=jax.ShapeDtypeStruct((T - 1, B), jnp.float32),
    scratch_types=[
        pltpu.VMEM((T * RW,), jnp.int32),
        pltpu.VMEM((RW, D), jnp.float32),
        pltpu.VMEM((RW, D), jnp.float32),
        pltpu.VMEM((RW, D), jnp.float32),
        pltpu.VMEM((T - 1, RW), jnp.float32),
        pltpu.SemaphoreType.DMA,
        pltpu.SemaphoreType.DMA,
        pltpu.SemaphoreType.DMA,
    ],
    compiler_params=pltpu.CompilerParams(
        use_tc_tiling_on_sc=False, needs_layout_passes=False),
)
def _sc_scores(emb_hbm, vemb_hbm, ids_hbm, out_hbm,
               idx_v, wbuf, vbuf0, vbuf1, sbuf, gw, g0, g1):
    wid = lax.axis_index("s") * _NC + lax.axis_index("c")
    vbufs = (vbuf0, vbuf1)
    gsems = (g0, g1)

    pltpu.sync_copy(ids_hbm.at[wid], idx_v)

    w_cp = pltpu.async_copy(emb_hbm.at[idx_v.at[pl.ds(0, RW)]], wbuf, gw)
    gathers = [None, None]
    gathers[0] = pltpu.async_copy(
        vemb_hbm.at[idx_v.at[pl.ds(RW, RW)]], vbufs[0], gsems[0])
    w_cp.wait()

    def compute_stream(t, vb):
        def group_body(g, _):
            rows = lax.iota(jnp.int32, _L) + g * _L
            acc = jnp.zeros((_L,), jnp.float32)
            cols0 = lax.iota(jnp.int32, _L)
            for d in range(D):
                cols = (cols0 + d) & (D - 1)
                wcol = plsc.load_gather(wbuf, [rows, cols])
                vcol = plsc.load_gather(vb, [rows, cols])
                acc = acc + wcol * vcol
            sbuf[t - 1, pl.ds(g * _L, _L)] = acc
            return 0

        lax.fori_loop(0, _GROUPS, group_body, 0)

    for t in range(1, T):
        cur = (t - 1) % 2
        if t + 1 < T:
            gathers[t % 2] = pltpu.async_copy(
                vemb_hbm.at[idx_v.at[pl.ds((t + 1) * RW, RW)]],
                vbufs[t % 2], gsems[t % 2])
        gathers[cur].wait()
        compute_stream(t, vbufs[cur])

    pltpu.sync_copy(sbuf, out_hbm.at[:, pl.ds(wid * RW, RW)])


_TW = 8192
_TGRID = -(-1000000 // _TW)


def _transpose_body(x_ref, o_ref):
    eye = jnp.eye(D, dtype=jnp.float32)
    o_ref[...] = jax.lax.dot_general(
        x_ref[...], eye, (((0,), (0,)), ((), ())),
        preferred_element_type=jnp.float32)


_tc_rowmajor = pl.pallas_call(
    _transpose_body,
    grid=(_TGRID,),
    in_specs=[pl.BlockSpec((D, _TW), lambda i: (0, i))],
    out_specs=pl.BlockSpec((_TW, D), lambda i: (i, 0)),
    out_shape=jax.ShapeDtypeStruct((1000000, D), jnp.float32),
)


def _loss_body(s_ref, out_ref):
    s = jnp.clip(s_ref[...], -10.0, 10.0)
    pos = s[0:1]
    neg = s[1:]
    tot = jnp.sum(jnp.log1p(jnp.exp(-pos))) + jnp.sum(jnp.log1p(jnp.exp(neg)))
    out_ref[...] = jnp.reshape(tot, (1, 1))


_loss = pl.pallas_call(
    _loss_body,
    out_shape=jax.ShapeDtypeStruct((1, 1), jnp.float32),
)


def kernel(embedding, v_embedding, wids, vids, neg_vids):
    ids = jnp.concatenate([
        wids.astype(jnp.int32),
        vids.astype(jnp.int32),
        neg_vids.T.astype(jnp.int32).reshape(-1),
    ])
    ids = ids.reshape(T, NW, RW).transpose(1, 0, 2).reshape(NW, -1)
    emb_rm = _tc_rowmajor(embedding.T)
    vemb_rm = _tc_rowmajor(v_embedding.T)
    scores = _sc_scores(emb_rm, vemb_rm, ids)
    return _loss(scores)[0, 0] / B


# speedup vs baseline: 1.6027x; 1.0387x over previous
import functools

import jax
import jax.numpy as jnp
from jax import lax
from jax.experimental import pallas as pl
from jax.experimental.pallas import tpu as pltpu
from jax.experimental.pallas import tpu_sc as plsc

D = 64
B = 16384
NNEG = 5
T = 2 + NNEG

_info = plsc.get_sparse_core_info()
_NC, _NS, _L = _info.num_cores, _info.num_subcores, _info.num_lanes
NW = _NC * _NS
RW = B // NW
_GROUPS = RW // _L

_mesh = plsc.VectorSubcoreMesh(core_axis_name="c", subcore_axis_name="s")


@functools.partial(
    pl.kernel,
    mesh=_mesh,
    out_type=jax.ShapeDtypeStruct((T - 1, B), jnp.float32),
    scratch_types=[
        pltpu.VMEM((T * RW,), jnp.int32),
        pltpu.VMEM((RW, D), jnp.float32),
        pltpu.VMEM((RW, D), jnp.float32),
        pltpu.VMEM((RW, D), jnp.float32),
        pltpu.VMEM((T - 1, RW), jnp.float32),
        pltpu.SemaphoreType.DMA,
        pltpu.SemaphoreType.DMA,
        pltpu.SemaphoreType.DMA,
    ],
    compiler_params=pltpu.CompilerParams(
        use_tc_tiling_on_sc=False, needs_layout_passes=False),
)
def _sc_scores(emb_hbm, vemb_hbm, ids_hbm, out_hbm,
               idx_v, wbuf, vbuf0, vbuf1, sbuf, gw, g0, g1):
    wid = lax.axis_index("s") * _NC + lax.axis_index("c")
    vbufs = (vbuf0, vbuf1)
    gsems = (g0, g1)

    pltpu.sync_copy(ids_hbm.at[wid], idx_v)

    w_cp = pltpu.async_copy(emb_hbm.at[idx_v.at[pl.ds(0, RW)]], wbuf, gw)
    gathers = [None, None]
    gathers[0] = pltpu.async_copy(
        vemb_hbm.at[idx_v.at[pl.ds(RW, RW)]], vbufs[0], gsems[0])
    w_cp.wait()

    def compute_stream(t, vb):
        def group_body(g, _):
            rows = lax.iota(jnp.int32, _L) + g * _L
            acc = jnp.zeros((_L,), jnp.float32)
            cols0 = lax.iota(jnp.int32, _L)
            for d in range(D):
                cols = (cols0 + d) & (D - 1)
                wcol = plsc.load_gather(wbuf, [rows, cols])
                vcol = plsc.load_gather(vb, [rows, cols])
                acc = acc + wcol * vcol
            sbuf[t - 1, pl.ds(g * _L, _L)] = acc
            return 0

        lax.fori_loop(0, _GROUPS, group_body, 0)

    for t in range(1, T):
        cur = (t - 1) % 2
        if t + 1 < T:
            gathers[t % 2] = pltpu.async_copy(
                vemb_hbm.at[idx_v.at[pl.ds((t + 1) * RW, RW)]],
                vbufs[t % 2], gsems[t % 2])
        gathers[cur].wait()
        compute_stream(t, vbufs[cur])

    pltpu.sync_copy(sbuf, out_hbm.at[:, pl.ds(wid * RW, RW)])


_TW = 16384
_TGRID = -(-1000000 // _TW)


def _transpose_body(x_ref, o_ref):
    eye = jnp.eye(D, dtype=jnp.float32)
    o_ref[...] = jax.lax.dot_general(
        x_ref[...], eye, (((0,), (0,)), ((), ())),
        preferred_element_type=jnp.float32)


_tc_rowmajor = pl.pallas_call(
    _transpose_body,
    grid=(_TGRID,),
    in_specs=[pl.BlockSpec((D, _TW), lambda i: (0, i))],
    out_specs=pl.BlockSpec((_TW, D), lambda i: (i, 0)),
    out_shape=jax.ShapeDtypeStruct((1000000, D), jnp.float32),
)


def _loss_body(s_ref, out_ref):
    s = jnp.clip(s_ref[...], -10.0, 10.0)
    pos = s[0:1]
    neg = s[1:]
    tot = jnp.sum(jnp.log1p(jnp.exp(-pos))) + jnp.sum(jnp.log1p(jnp.exp(neg)))
    out_ref[...] = jnp.reshape(tot, (1, 1))


_loss = pl.pallas_call(
    _loss_body,
    out_shape=jax.ShapeDtypeStruct((1, 1), jnp.float32),
)


def kernel(embedding, v_embedding, wids, vids, neg_vids):
    ids = jnp.concatenate([
        wids.astype(jnp.int32),
        vids.astype(jnp.int32),
        neg_vids.T.astype(jnp.int32).reshape(-1),
    ])
    ids = ids.reshape(T, NW, RW).transpose(1, 0, 2).reshape(NW, -1)
    emb_rm = _tc_rowmajor(embedding.T)
    vemb_rm = _tc_rowmajor(v_embedding.T)
    scores = _sc_scores(emb_rm, vemb_rm, ids)
    return _loss(scores)[0, 0] / B


# speedup vs baseline: 1.6837x; 1.0506x over previous
import functools

import jax
import jax.numpy as jnp
from jax import lax
from jax.experimental import pallas as pl
from jax.experimental.pallas import tpu as pltpu
from jax.experimental.pallas import tpu_sc as plsc

D = 64
B = 16384
NNEG = 5
T = 2 + NNEG

_info = plsc.get_sparse_core_info()
_NC, _NS, _L = _info.num_cores, _info.num_subcores, _info.num_lanes
NW = _NC * _NS
RW = B // NW
_GROUPS = RW // _L

_mesh = plsc.VectorSubcoreMesh(core_axis_name="c", subcore_axis_name="s")


@functools.partial(
    pl.kernel,
    mesh=_mesh,
    out_type=jax.ShapeDtypeStruct((T - 1, B), jnp.float32),
    scratch_types=[
        pltpu.VMEM((T * RW,), jnp.int32),
        pltpu.VMEM((RW, D), jnp.float32),
        pltpu.VMEM((RW, D), jnp.float32),
        pltpu.VMEM((RW, D), jnp.float32),
        pltpu.VMEM((T - 1, RW), jnp.float32),
        pltpu.SemaphoreType.DMA,
        pltpu.SemaphoreType.DMA,
        pltpu.SemaphoreType.DMA,
    ],
    compiler_params=pltpu.CompilerParams(
        use_tc_tiling_on_sc=False, needs_layout_passes=False),
)
def _sc_scores(emb_hbm, vemb_hbm, ids_hbm, out_hbm,
               idx_v, wbuf, vbuf0, vbuf1, sbuf, gw, g0, g1):
    wid = lax.axis_index("s") * _NC + lax.axis_index("c")
    vbufs = (vbuf0, vbuf1)
    gsems = (g0, g1)

    pltpu.sync_copy(ids_hbm.at[wid], idx_v)

    w_cp = pltpu.async_copy(emb_hbm.at[idx_v.at[pl.ds(0, RW)]], wbuf, gw)
    gathers = [None, None]
    gathers[0] = pltpu.async_copy(
        vemb_hbm.at[idx_v.at[pl.ds(RW, RW)]], vbufs[0], gsems[0])
    w_cp.wait()

    def compute_stream(t, vb):
        def group_body(g, _):
            rows = lax.iota(jnp.int32, _L) + g * _L
            acc = jnp.zeros((_L,), jnp.float32)
            cols0 = lax.iota(jnp.int32, _L)
            for d in range(D):
                cols = (cols0 + d) & (D - 1)
                wcol = plsc.load_gather(wbuf, [rows, cols])
                vcol = plsc.load_gather(vb, [rows, cols])
                acc = acc + wcol * vcol
            sbuf[t - 1, pl.ds(g * _L, _L)] = acc
            return 0

        lax.fori_loop(0, _GROUPS, group_body, 0)

    for t in range(1, T):
        cur = (t - 1) % 2
        if t + 1 < T:
            gathers[t % 2] = pltpu.async_copy(
                vemb_hbm.at[idx_v.at[pl.ds((t + 1) * RW, RW)]],
                vbufs[t % 2], gsems[t % 2])
        gathers[cur].wait()
        compute_stream(t, vbufs[cur])

    pltpu.sync_copy(sbuf, out_hbm.at[:, pl.ds(wid * RW, RW)])


_TW = 16384
_TGRID = -(-1000000 // _TW)


def _transpose_body(x_ref, o_ref):
    eye = jnp.eye(D, dtype=jnp.float32)
    o_ref[...] = jax.lax.dot_general(
        x_ref[...], eye, (((0,), (0,)), ((), ())),
        preferred_element_type=jnp.float32)


_tc_rowmajor = pl.pallas_call(
    _transpose_body,
    grid=(_TGRID,),
    in_specs=[pl.BlockSpec((D, _TW), lambda i: (0, i))],
    out_specs=pl.BlockSpec((_TW, D), lambda i: (i, 0)),
    out_shape=jax.ShapeDtypeStruct((1000000, D), jnp.float32),
)


def _loss_body(s_ref, out_ref):
    s = jnp.clip(s_ref[...], -10.0, 10.0)
    pos = s[0:1]
    neg = s[1:]
    tot = jnp.sum(jnp.log1p(jnp.exp(-pos))) + jnp.sum(jnp.log1p(jnp.exp(neg)))
    out_ref[...] = jnp.reshape(tot, (1, 1))


_loss = pl.pallas_call(
    _loss_body,
    out_shape=jax.ShapeDtypeStruct((1, 1), jnp.float32),
)


def kernel(embedding, v_embedding, wids, vids, neg_vids):
    ids = jnp.concatenate([
        wids.astype(jnp.int32),
        vids.astype(jnp.int32),
        neg_vids.T.astype(jnp.int32).reshape(-1),
    ])
    ids = ids.reshape(T, NW, RW).transpose(1, 0, 2).reshape(NW, -1)
    emb_rm = _tc_rowmajor(embedding.T)
    scores = _sc_scores(emb_rm, v_embedding, ids)
    return _loss(scores)[0, 0] / B


# speedup vs baseline: 1.8477x; 1.0974x over previous
import functools

import jax
import jax.numpy as jnp
from jax import lax
from jax.experimental import pallas as pl
from jax.experimental.pallas import tpu as pltpu
from jax.experimental.pallas import tpu_sc as plsc

D = 64
B = 16384
NNEG = 5
T = 2 + NNEG

_info = plsc.get_sparse_core_info()
_NC, _NS, _L = _info.num_cores, _info.num_subcores, _info.num_lanes
NW = _NC * _NS
RW = B // NW
_GROUPS = RW // _L

_mesh = plsc.VectorSubcoreMesh(core_axis_name="c", subcore_axis_name="s")


@functools.partial(
    pl.kernel,
    mesh=_mesh,
    out_type=jax.ShapeDtypeStruct((T - 1, B), jnp.float32),
    scratch_types=[
        pltpu.VMEM((T * RW,), jnp.int32),
        pltpu.VMEM((RW, D), jnp.float32),
        pltpu.VMEM((RW, D), jnp.float32),
        pltpu.VMEM((RW, D), jnp.float32),
        pltpu.VMEM((T - 1, RW), jnp.float32),
        pltpu.SemaphoreType.DMA,
        pltpu.SemaphoreType.DMA,
        pltpu.SemaphoreType.DMA,
    ],
    compiler_params=pltpu.CompilerParams(
        use_tc_tiling_on_sc=False, needs_layout_passes=False),
)
def _sc_scores(emb_hbm, vemb_hbm, ids_hbm, out_hbm,
               idx_v, wbuf, vbuf0, vbuf1, sbuf, gw, g0, g1):
    wid = lax.axis_index("s") * _NC + lax.axis_index("c")
    vbufs = (vbuf0, vbuf1)
    gsems = (g0, g1)

    pltpu.sync_copy(ids_hbm.at[wid], idx_v)

    w_cp = pltpu.async_copy(emb_hbm.at[idx_v.at[pl.ds(0, RW)]], wbuf, gw)
    gathers = [None, None]
    gathers[0] = pltpu.async_copy(
        vemb_hbm.at[idx_v.at[pl.ds(RW, RW)]], vbufs[0], gsems[0])
    w_cp.wait()

    def compute_stream(t, vb):
        def group_body(g, _):
            rows = lax.iota(jnp.int32, _L) + g * _L
            acc = jnp.zeros((_L,), jnp.float32)
            cols0 = lax.iota(jnp.int32, _L)
            for d in range(D):
                cols = (cols0 + d) & (D - 1)
                wcol = plsc.load_gather(wbuf, [rows, cols])
                vcol = plsc.load_gather(vb, [rows, cols])
                acc = acc + wcol * vcol
            sbuf[t - 1, pl.ds(g * _L, _L)] = acc
            return 0

        lax.fori_loop(0, _GROUPS, group_body, 0)

    for t in range(1, T):
        cur = (t - 1) % 2
        if t + 1 < T:
            gathers[t % 2] = pltpu.async_copy(
                vemb_hbm.at[idx_v.at[pl.ds((t + 1) * RW, RW)]],
                vbufs[t % 2], gsems[t % 2])
        gathers[cur].wait()
        compute_stream(t, vbufs[cur])

    pltpu.sync_copy(sbuf, out_hbm.at[:, pl.ds(wid * RW, RW)])


def _loss_body(s_ref, out_ref):
    s = jnp.clip(s_ref[...], -10.0, 10.0)
    pos = s[0:1]
    neg = s[1:]
    tot = jnp.sum(jnp.log1p(jnp.exp(-pos))) + jnp.sum(jnp.log1p(jnp.exp(neg)))
    out_ref[...] = jnp.reshape(tot, (1, 1))


_loss = pl.pallas_call(
    _loss_body,
    out_shape=jax.ShapeDtypeStruct((1, 1), jnp.float32),
)


def kernel(embedding, v_embedding, wids, vids, neg_vids):
    ids = jnp.concatenate([
        wids.astype(jnp.int32),
        vids.astype(jnp.int32),
        neg_vids.T.astype(jnp.int32).reshape(-1),
    ])
    ids = ids.reshape(T, NW, RW).transpose(1, 0, 2).reshape(NW, -1)
    scores = _sc_scores(embedding, v_embedding, ids)
    return _loss(scores)[0, 0] / B
